# baseline (speedup 1.0000x reference)
import functools

import jax
import jax.numpy as jnp
from jax import lax
from jax.experimental import pallas as pl
from jax.experimental.pallas import tpu as pltpu
from jax.experimental.pallas import tpu_sc as plsc

D = 32
NBUF = 4


def _make(R, C, NC, NS):
    NW = NC * NS
    r_per_w = R // NW
    n_rounds = r_per_w // NBUF
    mesh = plsc.VectorSubcoreMesh(core_axis_name="c", subcore_axis_name="s")

    @functools.partial(
        pl.kernel,
        mesh=mesh,
        out_type=jax.ShapeDtypeStruct((R, C, D), jnp.float32),
        scratch_types=[
            pltpu.VMEM((r_per_w, C), jnp.int32),
            pltpu.VMEM((NBUF, C, D), jnp.float32),
        ]
        + [pltpu.SemaphoreType.DMA] * (2 * NBUF),
        compiler_params=pltpu.CompilerParams(use_tc_tiling_on_sc=False),
    )
    def emb(table_hbm, x_hbm, out_hbm, idx_v, rows_v, *sems):
        gsems = sems[:NBUF]
        wsems = sems[NBUF:]
        wid = lax.axis_index("s") * NC + lax.axis_index("c")
        base = wid * r_per_w
        pltpu.sync_copy(x_hbm.at[pl.ds(base, r_per_w)], idx_v)

        def fire_g(g, b):
            pltpu.async_copy(table_hbm.at[idx_v.at[g]], rows_v.at[b], gsems[b])

        def drain_g(b):
            pltpu.make_async_copy(
                table_hbm.at[idx_v.at[0]], rows_v.at[b], gsems[b]
            ).wait()

        def fire_w(g, b):
            pltpu.async_copy(rows_v.at[b], out_hbm.at[base + g], wsems[b])

        def drain_w(b):
            pltpu.make_async_copy(
                rows_v.at[b], out_hbm.at[base], wsems[b]
            ).wait()

        for b in range(NBUF):
            fire_g(b, b)

        def body(it, carry):
            g0 = it * NBUF
            for b in range(NBUF):
                g = g0 + b
                drain_g(b)
                fire_w(g, b)
                drain_w(b)
                fire_g(g + NBUF, b)
            return carry

        lax.fori_loop(0, n_rounds - 1, body, 0)

        g0 = (n_rounds - 1) * NBUF
        for b in range(NBUF):
            drain_g(b)
            fire_w(g0 + b, b)
        for b in range(NBUF):
            drain_w(b)

    return emb


def kernel(x, table):
    R, C = x.shape
    info = plsc.get_sparse_core_info()
    NC, NS = info.num_cores, info.num_subcores
    H = C // 2
    emb = _make(R, H, NC, NS)
    out_a = emb(table, x[:, :H])
    out_b = emb(table, x[:, H:])
    return jnp.concatenate([out_a, out_b], axis=1)


# speedup vs baseline: 1.0296x; 1.0296x over previous
import functools

import jax
import jax.numpy as jnp
from jax import lax
from jax.experimental import pallas as pl
from jax.experimental.pallas import tpu as pltpu
from jax.experimental.pallas import tpu_sc as plsc

D = 32
NBUF = 4


def kernel(x, table):
    R, C = x.shape
    info = plsc.get_sparse_core_info()
    NC, NS = info.num_cores, info.num_subcores
    NW = NC * NS
    r_per_w = R // NW
    n_rounds = r_per_w // NBUF

    mesh = plsc.VectorSubcoreMesh(core_axis_name="c", subcore_axis_name="s")

    @functools.partial(
        pl.kernel,
        mesh=mesh,
        out_type=jax.ShapeDtypeStruct((R, C, D), jnp.float32),
        scratch_types=[
            pltpu.VMEM((r_per_w, C), jnp.int32),
            pltpu.VMEM((NBUF, C, D), jnp.float32),
        ]
        + [pltpu.SemaphoreType.DMA] * (2 * NBUF),
        compiler_params=pltpu.CompilerParams(use_tc_tiling_on_sc=False),
    )
    def emb(table_hbm, x_hbm, out_hbm, idx_v, rows_v, *sems):
        gsems = sems[:NBUF]
        wsems = sems[NBUF:]
        wid = lax.axis_index("s") * NC + lax.axis_index("c")
        base = wid * r_per_w
        pltpu.sync_copy(x_hbm.at[pl.ds(base, r_per_w)], idx_v)

        def fire_g(g, b):
            pltpu.async_copy(table_hbm.at[idx_v.at[g]], rows_v.at[b], gsems[b])

        def drain_g(b):
            pltpu.make_async_copy(
                table_hbm.at[idx_v.at[0]], rows_v.at[b], gsems[b]
            ).wait()

        def fire_w(g, b):
            pltpu.async_copy(rows_v.at[b], out_hbm.at[base + g], wsems[b])

        def drain_w(b):
            pltpu.make_async_copy(
                rows_v.at[b], out_hbm.at[base], wsems[b]
            ).wait()

        for b in range(NBUF):
            fire_g(b, b)

        def body(it, carry):
            g0 = it * NBUF
            for b in range(NBUF):
                g = g0 + b
                drain_g(b)
                fire_w(g, b)
                drain_w(b)
                fire_g(g + NBUF, b)
            return carry

        lax.fori_loop(0, n_rounds - 1, body, 0)

        g0 = (n_rounds - 1) * NBUF
        for b in range(NBUF):
            drain_g(b)
            fire_w(g0 + b, b)
        for b in range(NBUF):
            drain_w(b)

    return emb(table, x)
